# baseline (speedup 1.0000x reference)
import functools

import jax
import jax.numpy as jnp
from jax import lax
from jax.experimental import pallas as pl
from jax.experimental.pallas import tpu as pltpu
from jax.experimental.pallas import tpu_sc as plsc

VOCAB = 23236
EMBED_DIM = 1728
BATCH = 32
SEQ = 512

_NC = 2
_NS = 16
_NW = _NC * _NS

_B = BATCH * SEQ
_BPW = _B // _NW
_K = 32
_NCH = _BPW // _K
_NFULL = EMBED_DIM // 128
_TAIL0 = _NFULL * 128


_SPROWS = 1472
_SCH = 32
_SFULL = _SPROWS // _SCH
_LAST0 = (_NS - 1) * _SPROWS
_LFULL = ((VOCAB // 8) * 8 - _LAST0) // _SCH
_TAILR0 = (VOCAB // 8) * 8
_TAILRN = VOCAB - _TAILR0


def _gather_body(table_hbm, ids_hbm, out_hbm, tstage_hbm, idx_v,
                 rows0, rows1, tail0, tail1, stg64,
                 t4_64, t4_128, gsem0, gsem1):
    cid = lax.axis_index("c")
    sid = lax.axis_index("s")
    wid = sid * _NC + cid
    base = wid * _BPW

    def stage(r0, s64, s128, n):
        r0 = pl.multiple_of(r0, 8)
        pltpu.sync_copy(
            table_hbm.at[pl.ds(r0, n), pl.ds(_TAIL0, 64)], s64)

        def srow(r, _):
            for k in range(4):
                s128[r, pl.ds(16 * k, 16)] = s64[r, pl.ds(16 * k, 16)]
            return _
        lax.fori_loop(0, n, srow, 0)
        pltpu.sync_copy(s128, tstage_hbm.at[pl.ds(r0, n)])

    tbase = pl.multiple_of(sid * _SPROWS, 8)

    @pl.when(sid < _NS - 1)
    def _():
        def sub(i, _):
            stage(tbase + i * _SCH, stg64, tail0, _SCH)
            return _
        lax.fori_loop(0, _SFULL, sub, 0)

    @pl.when(sid == _NS - 1)
    def _():
        def sub(i, _):
            stage(tbase + i * _SCH, stg64, tail0, _SCH)
            return _
        lax.fori_loop(0, _LFULL, sub, 0)
        stage(_TAILR0, t4_64, t4_128, _TAILRN)

    pltpu.sync_copy(ids_hbm.at[pl.ds(base, _BPW)], idx_v)

    plsc.subcore_barrier()

    def start_gathers(c, rows, tail, sem):
        idx = idx_v.at[pl.ds(c * _K, _K)]
        for j in range(_NFULL):
            pltpu.async_copy(
                table_hbm.at[idx, pl.ds(j * 128, 128)],
                rows.at[:, pl.ds(j * 128, 128)], sem)
        pltpu.async_copy(tstage_hbm.at[idx], tail, sem)

    def wait_gathers(c, rows, tail, sem):
        idx = idx_v.at[pl.ds(c * _K, _K)]
        for j in range(_NFULL):
            pltpu.make_async_copy(
                table_hbm.at[idx, pl.ds(j * 128, 128)],
                rows.at[:, pl.ds(j * 128, 128)], sem).wait()
        pltpu.make_async_copy(tstage_hbm.at[idx], tail, sem).wait()

    start_gathers(0, rows0, tail0, gsem0)
    start_gathers(1, rows1, tail1, gsem1)

    def step(c, rows, tail, sem):
        wait_gathers(c, rows, tail, sem)

        def merge_row(r, _):
            for k in range(4):
                rows[r, pl.ds(_TAIL0 + 16 * k, 16)] = \
                    tail[r, pl.ds(16 * k, 16)]
            return _
        lax.fori_loop(0, _K, merge_row, 0)

        pltpu.sync_copy(rows, out_hbm.at[pl.ds(base + c * _K, _K)])

        @pl.when(c + 2 < _NCH)
        def _():
            start_gathers(c + 2, rows, tail, sem)

    def pair(i, _):
        step(2 * i, rows0, tail0, gsem0)
        step(2 * i + 1, rows1, tail1, gsem1)
        return _

    lax.fori_loop(0, _NCH // 2, pair, 0)


@jax.jit
def _embed(ids_flat, font_table):
    mesh = plsc.VectorSubcoreMesh(core_axis_name="c", subcore_axis_name="s")
    run = pl.kernel(
        _gather_body,
        out_type=(
            jax.ShapeDtypeStruct((_B, EMBED_DIM), jnp.float32),
            jax.ShapeDtypeStruct((VOCAB, 128), jnp.float32),
        ),
        mesh=mesh,
        scratch_types=[
            pltpu.VMEM((_BPW,), jnp.int32),
            pltpu.VMEM((_K, EMBED_DIM), jnp.float32),
            pltpu.VMEM((_K, EMBED_DIM), jnp.float32),
            pltpu.VMEM((_K, 128), jnp.float32),
            pltpu.VMEM((_K, 128), jnp.float32),
            pltpu.VMEM((_SCH, 64), jnp.float32),
            pltpu.VMEM((_TAILRN, 64), jnp.float32),
            pltpu.VMEM((_TAILRN, 128), jnp.float32),
            pltpu.SemaphoreType.DMA,
            pltpu.SemaphoreType.DMA,
        ],
    )
    out, _unused_stage = run(font_table, ids_flat)
    return out


def kernel(input_ids, font_table):
    ids_flat = input_ids.reshape(-1).astype(jnp.int32)
    out = _embed(ids_flat, font_table)
    return out.reshape(BATCH, SEQ, EMBED_DIM)
